# baseline (speedup 1.0000x reference)
import jax
import jax.numpy as jnp
import numpy as np
from jax import lax
from jax.experimental import pallas as pl
from jax.experimental.pallas import tpu as pltpu
from jax.experimental.pallas import tpu_sc as plsc
from functools import partial

_B = 8
_P = 2048
_Q = 1024
_KNN = 64


def _fps_body(S, pos_ref, posd_ref):
    px = pos_ref[0]
    py = pos_ref[1]
    pz = pos_ref[2]
    Pc = px.shape[1]
    sx = px[:, 0:1]
    sy = py[:, 0:1]
    sz = pz[:, 0:1]
    dx = px - sx
    dy = py - sy
    dz = pz - sz
    d = dx * dx + dy * dy + dz * dz
    iotaS = jax.lax.broadcasted_iota(jnp.int32, (_B, S), 1)
    big = jnp.int32(Pc)
    col0 = iotaS == 0
    ox = jnp.where(col0, sx, 0.0)
    oy = jnp.where(col0, sy, 0.0)
    oz = jnp.where(col0, sz, 0.0)
    NCH = Pc // 128
    jio = (
        jax.lax.broadcasted_iota(jnp.int32, (_B, NCH, 128), 1) * 128
        + jax.lax.broadcasted_iota(jnp.int32, (_B, NCH, 128), 2)
    )
    jio24 = (
        jax.lax.broadcasted_iota(jnp.int32, (3 * _B, NCH, 128), 1) * 128
        + jax.lax.broadcasted_iota(jnp.int32, (3 * _B, NCH, 128), 2)
    )
    stacked = jnp.concatenate([px, py, pz], axis=0).reshape(3 * _B, NCH, 128)

    def body(i, carry):
        d, ox, oy, oz = carry
        r3 = d.reshape(_B, NCH, 128)
        pm = jnp.max(r3, axis=1)
        m = jnp.max(pm, axis=1, keepdims=True)
        jsel = jnp.where(r3 == m[:, :, None], jio, big * 128)
        jmin = jnp.min(jnp.min(jsel, axis=1), axis=1, keepdims=True)
        j24 = jnp.concatenate([jmin, jmin, jmin], axis=0)
        ssel = jnp.where(jio24 == j24[:, :, None], stacked, -1.0)
        s = jnp.max(jnp.max(ssel, axis=1), axis=1, keepdims=True)
        sx = s[0:_B]
        sy = s[_B : 2 * _B]
        sz = s[2 * _B : 3 * _B]
        coli = iotaS == i
        ox = jnp.where(coli, sx, ox)
        oy = jnp.where(coli, sy, oy)
        oz = jnp.where(coli, sz, oz)
        dx = px - sx
        dy = py - sy
        dz = pz - sz
        d = jnp.minimum(d, dx * dx + dy * dy + dz * dz)
        return (d, ox, oy, oz)

    _, ox, oy, oz = jax.lax.fori_loop(1, S, body, (d, ox, oy, oz))
    posd_ref[0] = ox
    posd_ref[1] = oy
    posd_ref[2] = oz


def _fps_pallas(pos3, S):
    return pl.pallas_call(
        partial(_fps_body, S),
        out_shape=jax.ShapeDtypeStruct((3, _B, S), jnp.float32),
    )(pos3)


_NTILES = 32


def _sc_neighbors_body(Pc, S, dpt, r2bits, F, C, *refs):
    if F > 0:
        (px_h, py_h, pz_h, dx_h, dy_h, dz_h, ft_h, out_h,
         pxv, pyv, pzv, dxv, dyv, dzv, ftv, cd2, cidx, nbr, stg) = refs
    else:
        (px_h, py_h, pz_h, dx_h, dy_h, dz_h, out_h,
         pxv, pyv, pzv, dxv, dyv, dzv, cd2, cidx, nbr, stg) = refs
    wid = lax.axis_index("s") * 2 + lax.axis_index("c")
    dbase = wid * dpt
    b = dbase // S
    pltpu.sync_copy(px_h.at[pl.ds(b * Pc, Pc)], pxv)
    pltpu.sync_copy(py_h.at[pl.ds(b * Pc, Pc)], pyv)
    pltpu.sync_copy(pz_h.at[pl.ds(b * Pc, Pc)], pzv)
    pltpu.sync_copy(dx_h.at[pl.ds(dbase, dpt)], dxv)
    pltpu.sync_copy(dy_h.at[pl.ds(dbase, dpt)], dyv)
    pltpu.sync_copy(dz_h.at[pl.ds(dbase, dpt)], dzv)
    if F > 0:
        pltpu.sync_copy(ft_h.at[pl.ds(b * Pc * F, Pc * F)], ftv)
    z16i = jnp.zeros((16,), jnp.int32)
    z16f = jnp.zeros((16,), jnp.float32)
    for j in range(64 // 16):
        nbr[pl.ds(16 * j, 16)] = z16i

    def zinit(j, _):
        stg[pl.ds(pl.multiple_of(j * 16, 16), 16)] = z16f
        return jnp.int32(0)

    lax.fori_loop(0, 8 * 64 * C // 16, zinit, jnp.int32(0))
    lane = lax.broadcasted_iota(jnp.int32, (16,), 0)
    tru16 = lane < 16
    inf16 = jnp.full((16,), jnp.inf, jnp.float32)
    r2 = jnp.full((16,), float(np.int32(r2bits).view(np.float32)), jnp.float32)

    def dst_body(dl, _):
        dlv = jnp.zeros((16,), jnp.int32) + dl
        sxv = plsc.load_gather(dxv, [dlv])
        syv = plsc.load_gather(dyv, [dlv])
        szv = plsc.load_gather(dzv, [dlv])

        def chunk1(k, cntv):
            for u in range(2):
                st = pl.multiple_of(k * 32 + u * 16, 16)
                cx = pxv[pl.ds(st, 16)] - sxv
                cy = pyv[pl.ds(st, 16)] - syv
                cz = pzv[pl.ds(st, 16)] - szv
                d2 = cx * cx + cy * cy + cz * cz
                m = d2 <= r2
                off = cntv + plsc.cumsum(jnp.where(m, 1, 0)) - 1
                plsc.store_scatter(cd2, [off], d2, mask=m)
                plsc.store_scatter(cidx, [off], st + lane, mask=m)
                cntv = cntv + plsc.all_reduce_population_count(m)
            return cntv

        cntv = lax.fori_loop(0, Pc // 32, chunk1, jnp.zeros((16,), jnp.int32))
        cnt = cntv[0]
        plsc.store_compressed(cd2.at[pl.ds(cnt, 16)], inf16, mask=tru16)
        plsc.store_compressed(cd2.at[pl.ds(cnt + 16, 16)], inf16, mask=tru16)
        ncap = jnp.minimum(cnt, 64)
        nch = (cnt + 15) // 16
        nch2 = (cnt + 31) // 32

        def search():
            def wbody(_, c):
                lo, hi = c
                mid = (lo + hi) // 2

                def cchunk(k, acc):
                    for u in range(2):
                        st = pl.multiple_of(k * 32 + u * 16, 16)
                        bits = plsc.bitcast(cd2[pl.ds(st, 16)], jnp.int32)
                        acc = acc + plsc.all_reduce_population_count(bits <= mid)
                    return acc

                c64 = lax.fori_loop(0, nch2, cchunk, jnp.zeros((16,), jnp.int32))
                ok = c64 >= 64
                return (jnp.where(ok, lo, mid + 1), jnp.where(ok, mid, hi))

            lo, _ = lax.fori_loop(
                0, 31, wbody,
                (jnp.zeros((16,), jnp.int32), jnp.zeros((16,), jnp.int32) + r2bits),
            )

            def lchunk(k, acc):
                for u in range(2):
                    st = pl.multiple_of(k * 32 + u * 16, 16)
                    bits = plsc.bitcast(cd2[pl.ds(st, 16)], jnp.int32)
                    acc = acc + plsc.all_reduce_population_count(bits < lo)
                return acc

            cl = lax.fori_loop(0, nch2, lchunk, jnp.zeros((16,), jnp.int32))
            return lo, 64 - cl

        theta, need = lax.cond(
            cnt > 64,
            search,
            lambda: (
                jnp.zeros((16,), jnp.int32) + (r2bits + 1),
                jnp.zeros((16,), jnp.int32) + 64,
            ),
        )

        def fchunk(k, carry):
            posv, trankv = carry
            st = pl.multiple_of(k * 16, 16)
            bits = plsc.bitcast(cd2[pl.ds(st, 16)], jnp.int32)
            less = bits < theta
            tie = bits == theta
            tcs = plsc.cumsum(jnp.where(tie, 1, 0))
            sel = less | (tie & ((trankv + tcs) <= need))
            off = posv + plsc.cumsum(jnp.where(sel, 1, 0)) - 1
            plsc.store_scatter(nbr, [off], cidx[pl.ds(st, 16)], mask=sel)
            return (
                posv + plsc.all_reduce_population_count(sel),
                trankv + plsc.all_reduce_population_count(tie),
            )

        lax.fori_loop(
            0, nch, fchunk,
            (jnp.zeros((16,), jnp.int32), jnp.zeros((16,), jnp.int32)),
        )

        soff = (dl % 8) * (64 * C)
        for j in range(4):
            idxj = nbr[pl.ds(16 * j, 16)]
            rx = plsc.load_gather(pxv, [idxj]) - sxv
            ry = plsc.load_gather(pyv, [idxj]) - syv
            rz = plsc.load_gather(pzv, [idxj]) - szv
            vf = jnp.where((j * 16 + lane) < ncap, 1.0, 0.0)
            rowoff = soff + (j * 16 + lane) * C
            for c in range(F):
                fvals = plsc.load_gather(ftv, [idxj * F + c])
                plsc.store_scatter(stg, [rowoff + c], fvals)
            plsc.store_scatter(stg, [rowoff + F], rx)
            plsc.store_scatter(stg, [rowoff + F + 1], ry)
            plsc.store_scatter(stg, [rowoff + F + 2], rz)
            plsc.store_scatter(stg, [rowoff + F + 3], vf)

        @pl.when(dl % 8 == 7)
        def _():
            pltpu.sync_copy(
                stg, out_h.at[pl.ds((dbase + dl - 7) * 64 * C, 8 * 64 * C)]
            )

        return jnp.int32(0)

    lax.fori_loop(0, dpt, dst_body, jnp.int32(0))


def _sc_neighbors(Pc, S, r, F, C, posx, posy, posz, dstx, dsty, dstz, feat=None):
    total_dst = _B * S
    dpt = total_dst // _NTILES
    r2bits = int(np.float32(r * r).view(np.int32))
    mesh = plsc.VectorSubcoreMesh(core_axis_name="c", subcore_axis_name="s")
    scratch = [
        pltpu.VMEM((Pc,), jnp.float32),
        pltpu.VMEM((Pc,), jnp.float32),
        pltpu.VMEM((Pc,), jnp.float32),
        pltpu.VMEM((dpt,), jnp.float32),
        pltpu.VMEM((dpt,), jnp.float32),
        pltpu.VMEM((dpt,), jnp.float32),
    ]
    if F > 0:
        scratch.append(pltpu.VMEM((Pc * F,), jnp.float32))
    scratch += [
        pltpu.VMEM((Pc + 48,), jnp.float32),
        pltpu.VMEM((Pc + 48,), jnp.int32),
        pltpu.VMEM((80,), jnp.int32),
        pltpu.VMEM((8 * 64 * C,), jnp.float32),
    ]
    fn = pl.kernel(
        partial(_sc_neighbors_body, Pc, S, dpt, r2bits, F, C),
        out_type=jax.ShapeDtypeStruct((total_dst * 64 * C,), jnp.float32),
        mesh=mesh,
        scratch_types=scratch,
        compiler_params=pltpu.CompilerParams(needs_layout_passes=False),
    )
    args = (posx, posy, posz, dstx, dsty, dstz)
    if F > 0:
        args = args + (feat,)
    return fn(*args)


def _edge_mlp_body(F, C, Cout, msg_ref, w0_ref, b0_ref, w1_ref, b1_ref, w2_ref, b2_ref, out_ref):
    x = msg_ref[...]
    v = x[:, F + 3 : F + 4]
    h = jnp.dot(x, w0_ref[...], preferred_element_type=jnp.float32) + b0_ref[...]
    h = jnp.maximum(h, 0.0)
    h = jnp.dot(h, w1_ref[...], preferred_element_type=jnp.float32) + b1_ref[...]
    h = jnp.maximum(h, 0.0)
    h = jnp.dot(h, w2_ref[...], preferred_element_type=jnp.float32) + b2_ref[...]
    h = jnp.where(v > 0.5, h, -jnp.inf)
    te = h.shape[0]
    out_ref[...] = jnp.max(h.reshape(te // 64, 64, Cout), axis=1)


def _edge_mlp(msg, F, C, W0, b0, W1, b1, W2, b2, te):
    E = msg.shape[0]
    C1 = W1.shape[0]
    Cout = W2.shape[1]
    w0p = jnp.zeros((C, C1), jnp.float32).at[: F + 3].set(W0)
    grid = (E // te,)
    return pl.pallas_call(
        partial(_edge_mlp_body, F, C, Cout),
        grid=grid,
        in_specs=[
            pl.BlockSpec((te, C), lambda i: (i, 0)),
            pl.BlockSpec((C, C1), lambda i: (0, 0)),
            pl.BlockSpec((C1,), lambda i: (0,)),
            pl.BlockSpec((C1, C1), lambda i: (0, 0)),
            pl.BlockSpec((C1,), lambda i: (0,)),
            pl.BlockSpec((C1, Cout), lambda i: (0, 0)),
            pl.BlockSpec((Cout,), lambda i: (0,)),
        ],
        out_specs=pl.BlockSpec((te // 64, Cout), lambda i: (i, 0)),
        out_shape=jax.ShapeDtypeStruct((E // 64, Cout), jnp.float32),
    )(msg, w0p, b0, W1, b1, W2, b2)


def _mlp(x, Ws, bs):
    n = len(Ws)
    for j in range(n):
        x = x @ Ws[j] + bs[j]
        if j < n - 1:
            x = jax.nn.relu(x)
    return x


def _fps(pos, S):
    sel0 = jnp.zeros((S,), dtype=jnp.int32)
    d0 = jnp.sum((pos - pos[0]) ** 2, axis=1)

    def body(i, state):
        sel, d = state
        nxt = jnp.argmax(d).astype(jnp.int32)
        sel = sel.at[i].set(nxt)
        d = jnp.minimum(d, jnp.sum((pos - pos[nxt]) ** 2, axis=1))
        return (sel, d)

    sel, _ = jax.lax.fori_loop(1, S, body, (sel0, d0))
    return sel


def _sa_module(x, pos, Pc, S, r, Ws, bs, te):
    pos_c = pos.reshape(_B, Pc, 3)
    pos3 = jnp.transpose(pos_c, (2, 0, 1))
    posd = _fps_pallas(pos3, S)
    pos_dst = jnp.transpose(posd, (1, 2, 0)).reshape(_B * S, 3)
    F = 0 if x is None else x.shape[1]
    C = 8 if x is None else 80
    feat = None if x is None else x.reshape(-1)
    msg = _sc_neighbors(
        Pc, S, r, F, C,
        pos3[0].reshape(-1), pos3[1].reshape(-1), pos3[2].reshape(-1),
        posd[0].reshape(-1), posd[1].reshape(-1), posd[2].reshape(-1),
        feat,
    ).reshape(_B * S * _KNN, C)
    agg = _edge_mlp(msg, F, C, Ws[0], bs[0], Ws[1], bs[1], Ws[2], bs[2], te)
    return agg, pos_dst


def _tail_body(x2_ref, pos2_ref, qp_ref, eps_ref,
               w0a_ref, w0b_ref, b0_ref, w1_ref, b1_ref, w2_ref, b2_ref,
               muw_ref, mub_ref, lvw_ref, lvb_ref,
               d0q_ref, d0s_ref, d0c_ref, d0z_ref,
               d1_ref, d2_ref, d3_ref, d4_ref,
               hq_ref, mu_ref, lv_ref):
    x2 = x2_ref[0]
    pos2 = pos2_ref[0]
    h = (
        jnp.dot(x2, w0a_ref[...], preferred_element_type=jnp.float32)
        + jnp.dot(pos2, w0b_ref[...], preferred_element_type=jnp.float32)
        + b0_ref[...]
    )
    h = jnp.maximum(h, 0.0)
    h = jnp.dot(h, w1_ref[...], preferred_element_type=jnp.float32) + b1_ref[...]
    h = jnp.maximum(h, 0.0)
    h = jnp.dot(h, w2_ref[...], preferred_element_type=jnp.float32) + b2_ref[...]
    g = jnp.max(h, axis=0, keepdims=True)
    mu = jnp.dot(g, muw_ref[...], preferred_element_type=jnp.float32) + mub_ref[...]
    lv = jnp.dot(g, lvw_ref[...], preferred_element_type=jnp.float32) + lvb_ref[...]
    mu_ref[0] = mu
    lv_ref[0] = lv
    z = mu + eps_ref[0] * jnp.exp(0.5 * lv)
    qp = qp_ref[0]
    hz = jnp.dot(z, d0z_ref[...], preferred_element_type=jnp.float32)
    hq = (
        jnp.dot(qp, d0q_ref[...], preferred_element_type=jnp.float32)
        + jnp.dot(jnp.sin(qp), d0s_ref[...], preferred_element_type=jnp.float32)
        + jnp.dot(jnp.cos(qp), d0c_ref[...], preferred_element_type=jnp.float32)
        + hz
    )
    hq = jnp.maximum(hq, 0.0)
    hq = jnp.maximum(jnp.dot(hq, d1_ref[...], preferred_element_type=jnp.float32), 0.0)
    hq = jnp.maximum(jnp.dot(hq, d2_ref[...], preferred_element_type=jnp.float32), 0.0)
    hq = jnp.maximum(jnp.dot(hq, d3_ref[...], preferred_element_type=jnp.float32), 0.0)
    hq_ref[0] = jnp.dot(hq, d4_ref[...], preferred_element_type=jnp.float32)


def _tail_pallas(x2, pos2, qp, eps, sa3_W, sa3_b, mu_W, mu_b, lv_W, lv_b, dec_W):
    S2 = _P // 8
    full = lambda shape: pl.BlockSpec(shape, lambda i: tuple(0 for _ in shape))
    w0a = sa3_W[0][:128]
    w0b = sa3_W[0][128:131]
    d0q = dec_W[0][0:3]
    d0s = dec_W[0][3:6]
    d0c = dec_W[0][6:9]
    d0z = dec_W[0][9:41]
    return pl.pallas_call(
        _tail_body,
        grid=(_B,),
        in_specs=[
            pl.BlockSpec((1, S2, 128), lambda i: (i, 0, 0)),
            pl.BlockSpec((1, S2, 3), lambda i: (i, 0, 0)),
            pl.BlockSpec((1, _Q, 3), lambda i: (i, 0, 0)),
            pl.BlockSpec((1, 1, 32), lambda i: (i, 0, 0)),
            full((128, 128)), full((3, 128)), full((128,)),
            full((128, 256)), full((256,)),
            full((256, 512)), full((512,)),
            full((512, 32)), full((32,)),
            full((512, 32)), full((32,)),
            full((3, 64)), full((3, 64)), full((3, 64)), full((32, 64)),
            full((64, 64)), full((64, 64)), full((64, 64)), full((64, 1)),
        ],
        out_specs=[
            pl.BlockSpec((1, _Q, 1), lambda i: (i, 0, 0)),
            pl.BlockSpec((1, 1, 32), lambda i: (i, 0, 0)),
            pl.BlockSpec((1, 1, 32), lambda i: (i, 0, 0)),
        ],
        out_shape=[
            jax.ShapeDtypeStruct((_B, _Q, 1), jnp.float32),
            jax.ShapeDtypeStruct((_B, 1, 32), jnp.float32),
            jax.ShapeDtypeStruct((_B, 1, 32), jnp.float32),
        ],
    )(
        x2.reshape(_B, S2, 128), pos2.reshape(_B, S2, 3), qp, eps.reshape(_B, 1, 32),
        w0a, w0b, sa3_b[0], sa3_W[1], sa3_b[1], sa3_W[2], sa3_b[2],
        mu_W, mu_b, lv_W, lv_b,
        d0q, d0s, d0c, d0z, dec_W[1], dec_W[2], dec_W[3], dec_W[4],
    )


def kernel(surface_points, batch, query_points, sa1_W0, sa1_b0, sa1_W1, sa1_b1, sa1_W2, sa1_b2, sa2_W0, sa2_b0, sa2_W1, sa2_b1, sa2_W2, sa2_b2, sa3_W0, sa3_b0, sa3_W1, sa3_b1, sa3_W2, sa3_b2, mu_W, mu_b, lv_W, lv_b, dec_W0, dec_W1, dec_W2, dec_W3, dec_W4):
    sp = surface_points
    qp = query_points
    x1, pos1 = _sa_module(None, sp, _P, _P // 2, 0.2, [sa1_W0, sa1_W1, sa1_W2], [sa1_b0, sa1_b1, sa1_b2], te=8192)
    x2, pos2 = _sa_module(x1, pos1, _P // 2, _P // 8, 0.4, [sa2_W0, sa2_W1, sa2_W2], [sa2_b0, sa2_b1, sa2_b2], te=4096)
    eps = jax.random.normal(jax.random.key(1), (_B, 32), dtype=jnp.float32)
    hq, mu, lv = _tail_pallas(
        x2, pos2, qp, eps,
        [sa3_W0, sa3_W1, sa3_W2], [sa3_b0, sa3_b1, sa3_b2],
        mu_W, mu_b, lv_W, lv_b,
        [dec_W0, dec_W1, dec_W2, dec_W3, dec_W4],
    )
    return (hq, mu.reshape(_B, 32), lv.reshape(_B, 32))


# speedup vs baseline: 1.0491x; 1.0491x over previous
import jax
import jax.numpy as jnp
import numpy as np
from jax import lax
from jax.experimental import pallas as pl
from jax.experimental.pallas import tpu as pltpu
from jax.experimental.pallas import tpu_sc as plsc
from functools import partial

_B = 8
_P = 2048
_Q = 1024
_KNN = 64


def _fps_body(S, pos_ref, posd_ref):
    px = pos_ref[0]
    py = pos_ref[1]
    pz = pos_ref[2]
    Pc = px.shape[1]
    sx = px[:, 0:1]
    sy = py[:, 0:1]
    sz = pz[:, 0:1]
    dx = px - sx
    dy = py - sy
    dz = pz - sz
    d = dx * dx + dy * dy + dz * dz
    iota = jax.lax.broadcasted_iota(jnp.int32, (_B, Pc), 1)
    iotaS = jax.lax.broadcasted_iota(jnp.int32, (_B, S), 1)
    big = jnp.int32(Pc)
    col0 = iotaS == 0
    ox = jnp.where(col0, sx, 0.0)
    oy = jnp.where(col0, sy, 0.0)
    oz = jnp.where(col0, sz, 0.0)

    def body(i, carry):
        d, ox, oy, oz = carry
        m = jnp.max(d, axis=1, keepdims=True)
        idxf = jnp.min(jnp.where(d == m, iota, big), axis=1, keepdims=True)
        onehot = (iota == idxf).astype(jnp.float32)
        sx = jnp.sum(px * onehot, axis=1, keepdims=True)
        sy = jnp.sum(py * onehot, axis=1, keepdims=True)
        sz = jnp.sum(pz * onehot, axis=1, keepdims=True)
        coli = iotaS == i
        ox = jnp.where(coli, sx, ox)
        oy = jnp.where(coli, sy, oy)
        oz = jnp.where(coli, sz, oz)
        dx = px - sx
        dy = py - sy
        dz = pz - sz
        d = jnp.minimum(d, dx * dx + dy * dy + dz * dz)
        return (d, ox, oy, oz)

    _, ox, oy, oz = jax.lax.fori_loop(1, S, body, (d, ox, oy, oz))
    posd_ref[0] = ox
    posd_ref[1] = oy
    posd_ref[2] = oz


def _fps_pallas(pos3, S):
    return pl.pallas_call(
        partial(_fps_body, S),
        out_shape=jax.ShapeDtypeStruct((3, _B, S), jnp.float32),
    )(pos3)


_NTILES = 32


def _sc_neighbors_body(Pc, S, dpt, r2bits, F, C, *refs):
    if F > 0:
        (px_h, py_h, pz_h, dx_h, dy_h, dz_h, ft_h, out_h,
         pxv, pyv, pzv, dxv, dyv, dzv, ftv, cd2, cidx, nbr, stg) = refs
    else:
        (px_h, py_h, pz_h, dx_h, dy_h, dz_h, out_h,
         pxv, pyv, pzv, dxv, dyv, dzv, cd2, cidx, nbr, stg) = refs
    wid = lax.axis_index("s") * 2 + lax.axis_index("c")
    dbase = wid * dpt
    b = dbase // S
    pltpu.sync_copy(px_h.at[pl.ds(b * Pc, Pc)], pxv)
    pltpu.sync_copy(py_h.at[pl.ds(b * Pc, Pc)], pyv)
    pltpu.sync_copy(pz_h.at[pl.ds(b * Pc, Pc)], pzv)
    pltpu.sync_copy(dx_h.at[pl.ds(dbase, dpt)], dxv)
    pltpu.sync_copy(dy_h.at[pl.ds(dbase, dpt)], dyv)
    pltpu.sync_copy(dz_h.at[pl.ds(dbase, dpt)], dzv)
    if F > 0:
        pltpu.sync_copy(ft_h.at[pl.ds(b * Pc * F, Pc * F)], ftv)
    z16i = jnp.zeros((16,), jnp.int32)
    z16f = jnp.zeros((16,), jnp.float32)
    for j in range(64 // 16):
        nbr[pl.ds(16 * j, 16)] = z16i

    def zinit(j, _):
        stg[pl.ds(pl.multiple_of(j * 16, 16), 16)] = z16f
        return jnp.int32(0)

    lax.fori_loop(0, 8 * 64 * C // 16, zinit, jnp.int32(0))
    lane = lax.broadcasted_iota(jnp.int32, (16,), 0)
    tru16 = lane < 16
    inf16 = jnp.full((16,), jnp.inf, jnp.float32)
    r2 = jnp.full((16,), float(np.int32(r2bits).view(np.float32)), jnp.float32)

    def dst_body(dl, _):
        dlv = jnp.zeros((16,), jnp.int32) + dl
        sxv = plsc.load_gather(dxv, [dlv])
        syv = plsc.load_gather(dyv, [dlv])
        szv = plsc.load_gather(dzv, [dlv])

        def chunk1(k, cntv):
            for u in range(2):
                st = pl.multiple_of(k * 32 + u * 16, 16)
                cx = pxv[pl.ds(st, 16)] - sxv
                cy = pyv[pl.ds(st, 16)] - syv
                cz = pzv[pl.ds(st, 16)] - szv
                d2 = cx * cx + cy * cy + cz * cz
                m = d2 <= r2
                off = cntv + plsc.cumsum(jnp.where(m, 1, 0)) - 1
                plsc.store_scatter(cd2, [off], d2, mask=m)
                plsc.store_scatter(cidx, [off], st + lane, mask=m)
                cntv = cntv + plsc.all_reduce_population_count(m)
            return cntv

        cntv = lax.fori_loop(0, Pc // 32, chunk1, jnp.zeros((16,), jnp.int32))
        cnt = cntv[0]
        plsc.store_compressed(cd2.at[pl.ds(cnt, 16)], inf16, mask=tru16)
        plsc.store_compressed(cd2.at[pl.ds(cnt + 16, 16)], inf16, mask=tru16)
        ncap = jnp.minimum(cnt, 64)
        nch = (cnt + 15) // 16
        nch2 = (cnt + 31) // 32

        def search():
            def wbody(_, c):
                lo, hi = c
                mid = (lo + hi) // 2

                def cchunk(k, acc):
                    for u in range(2):
                        st = pl.multiple_of(k * 32 + u * 16, 16)
                        bits = plsc.bitcast(cd2[pl.ds(st, 16)], jnp.int32)
                        acc = acc + plsc.all_reduce_population_count(bits <= mid)
                    return acc

                c64 = lax.fori_loop(0, nch2, cchunk, jnp.zeros((16,), jnp.int32))
                ok = c64 >= 64
                return (jnp.where(ok, lo, mid + 1), jnp.where(ok, mid, hi))

            lo, _ = lax.fori_loop(
                0, 31, wbody,
                (jnp.zeros((16,), jnp.int32), jnp.zeros((16,), jnp.int32) + r2bits),
            )

            def lchunk(k, acc):
                for u in range(2):
                    st = pl.multiple_of(k * 32 + u * 16, 16)
                    bits = plsc.bitcast(cd2[pl.ds(st, 16)], jnp.int32)
                    acc = acc + plsc.all_reduce_population_count(bits < lo)
                return acc

            cl = lax.fori_loop(0, nch2, lchunk, jnp.zeros((16,), jnp.int32))
            return lo, 64 - cl

        theta, need = lax.cond(
            cnt > 64,
            search,
            lambda: (
                jnp.zeros((16,), jnp.int32) + (r2bits + 1),
                jnp.zeros((16,), jnp.int32) + 64,
            ),
        )

        def fchunk(k, carry):
            posv, trankv = carry
            st = pl.multiple_of(k * 16, 16)
            bits = plsc.bitcast(cd2[pl.ds(st, 16)], jnp.int32)
            less = bits < theta
            tie = bits == theta
            tcs = plsc.cumsum(jnp.where(tie, 1, 0))
            sel = less | (tie & ((trankv + tcs) <= need))
            off = posv + plsc.cumsum(jnp.where(sel, 1, 0)) - 1
            plsc.store_scatter(nbr, [off], cidx[pl.ds(st, 16)], mask=sel)
            return (
                posv + plsc.all_reduce_population_count(sel),
                trankv + plsc.all_reduce_population_count(tie),
            )

        lax.fori_loop(
            0, nch, fchunk,
            (jnp.zeros((16,), jnp.int32), jnp.zeros((16,), jnp.int32)),
        )

        soff = (dl % 8) * (64 * C)
        for j in range(4):
            idxj = nbr[pl.ds(16 * j, 16)]
            rx = plsc.load_gather(pxv, [idxj]) - sxv
            ry = plsc.load_gather(pyv, [idxj]) - syv
            rz = plsc.load_gather(pzv, [idxj]) - szv
            vf = jnp.where((j * 16 + lane) < ncap, 1.0, 0.0)
            rowoff = soff + (j * 16 + lane) * C
            for c in range(F):
                fvals = plsc.load_gather(ftv, [idxj * F + c])
                plsc.store_scatter(stg, [rowoff + c], fvals)
            plsc.store_scatter(stg, [rowoff + F], rx)
            plsc.store_scatter(stg, [rowoff + F + 1], ry)
            plsc.store_scatter(stg, [rowoff + F + 2], rz)
            plsc.store_scatter(stg, [rowoff + F + 3], vf)

        @pl.when(dl % 8 == 7)
        def _():
            pltpu.sync_copy(
                stg, out_h.at[pl.ds((dbase + dl - 7) * 64 * C, 8 * 64 * C)]
            )

        return jnp.int32(0)

    lax.fori_loop(0, dpt, dst_body, jnp.int32(0))


def _sc_neighbors(Pc, S, r, F, C, posx, posy, posz, dstx, dsty, dstz, feat=None):
    total_dst = _B * S
    dpt = total_dst // _NTILES
    r2bits = int(np.float32(r * r).view(np.int32))
    mesh = plsc.VectorSubcoreMesh(core_axis_name="c", subcore_axis_name="s")
    scratch = [
        pltpu.VMEM((Pc,), jnp.float32),
        pltpu.VMEM((Pc,), jnp.float32),
        pltpu.VMEM((Pc,), jnp.float32),
        pltpu.VMEM((dpt,), jnp.float32),
        pltpu.VMEM((dpt,), jnp.float32),
        pltpu.VMEM((dpt,), jnp.float32),
    ]
    if F > 0:
        scratch.append(pltpu.VMEM((Pc * F,), jnp.float32))
    scratch += [
        pltpu.VMEM((Pc + 48,), jnp.float32),
        pltpu.VMEM((Pc + 48,), jnp.int32),
        pltpu.VMEM((80,), jnp.int32),
        pltpu.VMEM((8 * 64 * C,), jnp.float32),
    ]
    fn = pl.kernel(
        partial(_sc_neighbors_body, Pc, S, dpt, r2bits, F, C),
        out_type=jax.ShapeDtypeStruct((total_dst * 64 * C,), jnp.float32),
        mesh=mesh,
        scratch_types=scratch,
        compiler_params=pltpu.CompilerParams(needs_layout_passes=False),
    )
    args = (posx, posy, posz, dstx, dsty, dstz)
    if F > 0:
        args = args + (feat,)
    return fn(*args)


def _edge_mlp_body(F, C, Cout, msg_ref, w0_ref, b0_ref, w1_ref, b1_ref, w2_ref, b2_ref, out_ref):
    x = msg_ref[...]
    v = x[:, F + 3 : F + 4]
    h = jnp.dot(x, w0_ref[...], preferred_element_type=jnp.float32) + b0_ref[...]
    h = jnp.maximum(h, 0.0)
    h = jnp.dot(h, w1_ref[...], preferred_element_type=jnp.float32) + b1_ref[...]
    h = jnp.maximum(h, 0.0)
    h = jnp.dot(h, w2_ref[...], preferred_element_type=jnp.float32) + b2_ref[...]
    h = jnp.where(v > 0.5, h, -jnp.inf)
    te = h.shape[0]
    out_ref[...] = jnp.max(h.reshape(te // 64, 64, Cout), axis=1)


def _edge_mlp(msg, F, C, W0, b0, W1, b1, W2, b2, te):
    E = msg.shape[0]
    C1 = W1.shape[0]
    Cout = W2.shape[1]
    w0p = jnp.zeros((C, C1), jnp.float32).at[: F + 3].set(W0)
    grid = (E // te,)
    return pl.pallas_call(
        partial(_edge_mlp_body, F, C, Cout),
        grid=grid,
        in_specs=[
            pl.BlockSpec((te, C), lambda i: (i, 0)),
            pl.BlockSpec((C, C1), lambda i: (0, 0)),
            pl.BlockSpec((C1,), lambda i: (0,)),
            pl.BlockSpec((C1, C1), lambda i: (0, 0)),
            pl.BlockSpec((C1,), lambda i: (0,)),
            pl.BlockSpec((C1, Cout), lambda i: (0, 0)),
            pl.BlockSpec((Cout,), lambda i: (0,)),
        ],
        out_specs=pl.BlockSpec((te // 64, Cout), lambda i: (i, 0)),
        out_shape=jax.ShapeDtypeStruct((E // 64, Cout), jnp.float32),
    )(msg, w0p, b0, W1, b1, W2, b2)


def _mlp(x, Ws, bs):
    n = len(Ws)
    for j in range(n):
        x = x @ Ws[j] + bs[j]
        if j < n - 1:
            x = jax.nn.relu(x)
    return x


def _fps(pos, S):
    sel0 = jnp.zeros((S,), dtype=jnp.int32)
    d0 = jnp.sum((pos - pos[0]) ** 2, axis=1)

    def body(i, state):
        sel, d = state
        nxt = jnp.argmax(d).astype(jnp.int32)
        sel = sel.at[i].set(nxt)
        d = jnp.minimum(d, jnp.sum((pos - pos[nxt]) ** 2, axis=1))
        return (sel, d)

    sel, _ = jax.lax.fori_loop(1, S, body, (sel0, d0))
    return sel


def _sa_module(x, pos, Pc, S, r, Ws, bs, te):
    pos_c = pos.reshape(_B, Pc, 3)
    pos3 = jnp.transpose(pos_c, (2, 0, 1))
    posd = _fps_pallas(pos3, S)
    pos_dst = jnp.transpose(posd, (1, 2, 0)).reshape(_B * S, 3)
    F = 0 if x is None else x.shape[1]
    C = 8 if x is None else 80
    feat = None if x is None else x.reshape(-1)
    msg = _sc_neighbors(
        Pc, S, r, F, C,
        pos3[0].reshape(-1), pos3[1].reshape(-1), pos3[2].reshape(-1),
        posd[0].reshape(-1), posd[1].reshape(-1), posd[2].reshape(-1),
        feat,
    ).reshape(_B * S * _KNN, C)
    agg = _edge_mlp(msg, F, C, Ws[0], bs[0], Ws[1], bs[1], Ws[2], bs[2], te)
    return agg, pos_dst


def _tail_body(x2_ref, pos2_ref, qp_ref, eps_ref,
               w0a_ref, w0b_ref, b0_ref, w1_ref, b1_ref, w2_ref, b2_ref,
               muw_ref, mub_ref, lvw_ref, lvb_ref,
               d0q_ref, d0s_ref, d0c_ref, d0z_ref,
               d1_ref, d2_ref, d3_ref, d4_ref,
               hq_ref, mu_ref, lv_ref):
    x2 = x2_ref[0]
    pos2 = pos2_ref[0]
    h = (
        jnp.dot(x2, w0a_ref[...], preferred_element_type=jnp.float32)
        + jnp.dot(pos2, w0b_ref[...], preferred_element_type=jnp.float32)
        + b0_ref[...]
    )
    h = jnp.maximum(h, 0.0)
    h = jnp.dot(h, w1_ref[...], preferred_element_type=jnp.float32) + b1_ref[...]
    h = jnp.maximum(h, 0.0)
    h = jnp.dot(h, w2_ref[...], preferred_element_type=jnp.float32) + b2_ref[...]
    g = jnp.max(h, axis=0, keepdims=True)
    mu = jnp.dot(g, muw_ref[...], preferred_element_type=jnp.float32) + mub_ref[...]
    lv = jnp.dot(g, lvw_ref[...], preferred_element_type=jnp.float32) + lvb_ref[...]
    mu_ref[0] = mu
    lv_ref[0] = lv
    z = mu + eps_ref[0] * jnp.exp(0.5 * lv)
    qp = qp_ref[0]
    hz = jnp.dot(z, d0z_ref[...], preferred_element_type=jnp.float32)
    hq = (
        jnp.dot(qp, d0q_ref[...], preferred_element_type=jnp.float32)
        + jnp.dot(jnp.sin(qp), d0s_ref[...], preferred_element_type=jnp.float32)
        + jnp.dot(jnp.cos(qp), d0c_ref[...], preferred_element_type=jnp.float32)
        + hz
    )
    hq = jnp.maximum(hq, 0.0)
    hq = jnp.maximum(jnp.dot(hq, d1_ref[...], preferred_element_type=jnp.float32), 0.0)
    hq = jnp.maximum(jnp.dot(hq, d2_ref[...], preferred_element_type=jnp.float32), 0.0)
    hq = jnp.maximum(jnp.dot(hq, d3_ref[...], preferred_element_type=jnp.float32), 0.0)
    hq_ref[0] = jnp.dot(hq, d4_ref[...], preferred_element_type=jnp.float32)


def _tail_pallas(x2, pos2, qp, eps, sa3_W, sa3_b, mu_W, mu_b, lv_W, lv_b, dec_W):
    S2 = _P // 8
    full = lambda shape: pl.BlockSpec(shape, lambda i: tuple(0 for _ in shape))
    w0a = sa3_W[0][:128]
    w0b = sa3_W[0][128:131]
    d0q = dec_W[0][0:3]
    d0s = dec_W[0][3:6]
    d0c = dec_W[0][6:9]
    d0z = dec_W[0][9:41]
    return pl.pallas_call(
        _tail_body,
        grid=(_B,),
        in_specs=[
            pl.BlockSpec((1, S2, 128), lambda i: (i, 0, 0)),
            pl.BlockSpec((1, S2, 3), lambda i: (i, 0, 0)),
            pl.BlockSpec((1, _Q, 3), lambda i: (i, 0, 0)),
            pl.BlockSpec((1, 1, 32), lambda i: (i, 0, 0)),
            full((128, 128)), full((3, 128)), full((128,)),
            full((128, 256)), full((256,)),
            full((256, 512)), full((512,)),
            full((512, 32)), full((32,)),
            full((512, 32)), full((32,)),
            full((3, 64)), full((3, 64)), full((3, 64)), full((32, 64)),
            full((64, 64)), full((64, 64)), full((64, 64)), full((64, 1)),
        ],
        out_specs=[
            pl.BlockSpec((1, _Q, 1), lambda i: (i, 0, 0)),
            pl.BlockSpec((1, 1, 32), lambda i: (i, 0, 0)),
            pl.BlockSpec((1, 1, 32), lambda i: (i, 0, 0)),
        ],
        out_shape=[
            jax.ShapeDtypeStruct((_B, _Q, 1), jnp.float32),
            jax.ShapeDtypeStruct((_B, 1, 32), jnp.float32),
            jax.ShapeDtypeStruct((_B, 1, 32), jnp.float32),
        ],
    )(
        x2.reshape(_B, S2, 128), pos2.reshape(_B, S2, 3), qp, eps.reshape(_B, 1, 32),
        w0a, w0b, sa3_b[0], sa3_W[1], sa3_b[1], sa3_W[2], sa3_b[2],
        mu_W, mu_b, lv_W, lv_b,
        d0q, d0s, d0c, d0z, dec_W[1], dec_W[2], dec_W[3], dec_W[4],
    )


def kernel(surface_points, batch, query_points, sa1_W0, sa1_b0, sa1_W1, sa1_b1, sa1_W2, sa1_b2, sa2_W0, sa2_b0, sa2_W1, sa2_b1, sa2_W2, sa2_b2, sa3_W0, sa3_b0, sa3_W1, sa3_b1, sa3_W2, sa3_b2, mu_W, mu_b, lv_W, lv_b, dec_W0, dec_W1, dec_W2, dec_W3, dec_W4):
    sp = surface_points
    qp = query_points
    x1, pos1 = _sa_module(None, sp, _P, _P // 2, 0.2, [sa1_W0, sa1_W1, sa1_W2], [sa1_b0, sa1_b1, sa1_b2], te=8192)
    x2, pos2 = _sa_module(x1, pos1, _P // 2, _P // 8, 0.4, [sa2_W0, sa2_W1, sa2_W2], [sa2_b0, sa2_b1, sa2_b2], te=4096)
    eps = jax.random.normal(jax.random.key(1), (_B, 32), dtype=jnp.float32)
    hq, mu, lv = _tail_pallas(
        x2, pos2, qp, eps,
        [sa3_W0, sa3_W1, sa3_W2], [sa3_b0, sa3_b1, sa3_b2],
        mu_W, mu_b, lv_W, lv_b,
        [dec_W0, dec_W1, dec_W2, dec_W3, dec_W4],
    )
    return (hq, mu.reshape(_B, 32), lv.reshape(_B, 32))


# speedup vs baseline: 1.2321x; 1.1744x over previous
import jax
import jax.numpy as jnp
import numpy as np
from jax import lax
from jax.experimental import pallas as pl
from jax.experimental.pallas import tpu as pltpu
from jax.experimental.pallas import tpu_sc as plsc
from functools import partial

_B = 8
_P = 2048
_Q = 1024
_KNN = 64


def _fps_body(S, pos_ref, posd_ref):
    px = pos_ref[0]
    py = pos_ref[1]
    pz = pos_ref[2]
    Pc = px.shape[1]
    sx = px[:, 0:1]
    sy = py[:, 0:1]
    sz = pz[:, 0:1]
    dx = px - sx
    dy = py - sy
    dz = pz - sz
    d = dx * dx + dy * dy + dz * dz
    iota = jax.lax.broadcasted_iota(jnp.int32, (_B, Pc), 1)
    iotaS = jax.lax.broadcasted_iota(jnp.int32, (_B, S), 1)
    big = jnp.int32(Pc)
    col0 = iotaS == 0
    ox = jnp.where(col0, sx, 0.0)
    oy = jnp.where(col0, sy, 0.0)
    oz = jnp.where(col0, sz, 0.0)

    def body(i, carry):
        d, ox, oy, oz = carry
        m = jnp.max(d, axis=1, keepdims=True)
        idxf = jnp.min(jnp.where(d == m, iota, big), axis=1, keepdims=True)
        onehot = (iota == idxf).astype(jnp.float32)
        sx = jnp.sum(px * onehot, axis=1, keepdims=True)
        sy = jnp.sum(py * onehot, axis=1, keepdims=True)
        sz = jnp.sum(pz * onehot, axis=1, keepdims=True)
        coli = iotaS == i
        ox = jnp.where(coli, sx, ox)
        oy = jnp.where(coli, sy, oy)
        oz = jnp.where(coli, sz, oz)
        dx = px - sx
        dy = py - sy
        dz = pz - sz
        d = jnp.minimum(d, dx * dx + dy * dy + dz * dz)
        return (d, ox, oy, oz)

    _, ox, oy, oz = jax.lax.fori_loop(1, S, body, (d, ox, oy, oz))
    posd_ref[0] = ox
    posd_ref[1] = oy
    posd_ref[2] = oz


def _fps_pallas(pos3, S):
    return pl.pallas_call(
        partial(_fps_body, S),
        out_shape=jax.ShapeDtypeStruct((3, _B, S), jnp.float32),
    )(pos3)


_NTILES = 32


def _sc_neighbors_body(Pc, S, dpt, r2bits, F, C, *refs):
    if F > 0:
        (px_h, py_h, pz_h, dx_h, dy_h, dz_h, ft_h, out_h,
         pxv, pyv, pzv, dxv, dyv, dzv, ftv, cd2, cidx, cd2b, cidxb, nbr, stg) = refs
    else:
        (px_h, py_h, pz_h, dx_h, dy_h, dz_h, out_h,
         pxv, pyv, pzv, dxv, dyv, dzv, cd2, cidx, cd2b, cidxb, nbr, stg) = refs
    wid = lax.axis_index("s") * 2 + lax.axis_index("c")
    dbase = wid * dpt
    b = dbase // S
    pltpu.sync_copy(px_h.at[pl.ds(b * Pc, Pc)], pxv)
    pltpu.sync_copy(py_h.at[pl.ds(b * Pc, Pc)], pyv)
    pltpu.sync_copy(pz_h.at[pl.ds(b * Pc, Pc)], pzv)
    pltpu.sync_copy(dx_h.at[pl.ds(dbase, dpt)], dxv)
    pltpu.sync_copy(dy_h.at[pl.ds(dbase, dpt)], dyv)
    pltpu.sync_copy(dz_h.at[pl.ds(dbase, dpt)], dzv)
    if F > 0:
        pltpu.sync_copy(ft_h.at[pl.ds(b * Pc * F, Pc * F)], ftv)
    z16i = jnp.zeros((16,), jnp.int32)
    z16f = jnp.zeros((16,), jnp.float32)
    for j in range(64 // 16):
        nbr[pl.ds(16 * j, 16)] = z16i

    def zinit(j, _):
        stg[pl.ds(pl.multiple_of(j * 16, 16), 16)] = z16f
        return jnp.int32(0)

    lax.fori_loop(0, 8 * 64 * C // 16, zinit, jnp.int32(0))
    lane = lax.broadcasted_iota(jnp.int32, (16,), 0)
    tru16 = lane < 16
    inf16 = jnp.full((16,), jnp.inf, jnp.float32)
    r2 = jnp.full((16,), float(np.int32(r2bits).view(np.float32)), jnp.float32)

    def dst_body(dl, _):
        dg0 = 2 * dl
        dg1 = 2 * dl + 1
        dlv0 = jnp.zeros((16,), jnp.int32) + dg0
        dlv1 = jnp.zeros((16,), jnp.int32) + dg1
        sxv0 = plsc.load_gather(dxv, [dlv0])
        syv0 = plsc.load_gather(dyv, [dlv0])
        szv0 = plsc.load_gather(dzv, [dlv0])
        sxv1 = plsc.load_gather(dxv, [dlv1])
        syv1 = plsc.load_gather(dyv, [dlv1])
        szv1 = plsc.load_gather(dzv, [dlv1])

        def chunk1(k, carry):
            cntv0, cntv1 = carry
            for u in range(2):
                st = pl.multiple_of(k * 32 + u * 16, 16)
                cx = pxv[pl.ds(st, 16)]
                cy = pyv[pl.ds(st, 16)]
                cz = pzv[pl.ds(st, 16)]
                ax = cx - sxv0
                ay = cy - syv0
                az = cz - szv0
                bx = cx - sxv1
                by = cy - syv1
                bz = cz - szv1
                d2a = ax * ax + ay * ay + az * az
                d2b = bx * bx + by * by + bz * bz
                ma = d2a <= r2
                mb = d2b <= r2
                offa = cntv0 + plsc.cumsum(jnp.where(ma, 1, 0)) - 1
                offb = cntv1 + plsc.cumsum(jnp.where(mb, 1, 0)) - 1
                plsc.store_scatter(cd2, [offa], d2a, mask=ma)
                plsc.store_scatter(cidx, [offa], st + lane, mask=ma)
                plsc.store_scatter(cd2b, [offb], d2b, mask=mb)
                plsc.store_scatter(cidxb, [offb], st + lane, mask=mb)
                cntv0 = cntv0 + plsc.all_reduce_population_count(ma)
                cntv1 = cntv1 + plsc.all_reduce_population_count(mb)
            return (cntv0, cntv1)

        cntv0, cntv1 = lax.fori_loop(
            0, Pc // 32, chunk1,
            (jnp.zeros((16,), jnp.int32), jnp.zeros((16,), jnp.int32)),
        )

        def finish(dg, sxv, syv, szv, cd2x, cidxx, cntv):
            cnt = cntv[0]
            plsc.store_compressed(cd2x.at[pl.ds(cnt, 16)], inf16, mask=tru16)
            plsc.store_compressed(cd2x.at[pl.ds(cnt + 16, 16)], inf16, mask=tru16)
            ncap = jnp.minimum(cnt, 64)
            nch = (cnt + 15) // 16
            nch2 = (cnt + 31) // 32

            def search():
                def wbody(_, c):
                    lo, hi = c
                    mid = (lo + hi) // 2

                    def cchunk(k, acc):
                        for u in range(2):
                            st = pl.multiple_of(k * 32 + u * 16, 16)
                            bits = plsc.bitcast(cd2x[pl.ds(st, 16)], jnp.int32)
                            acc = acc + plsc.all_reduce_population_count(bits <= mid)
                        return acc

                    c64 = lax.fori_loop(0, nch2, cchunk, jnp.zeros((16,), jnp.int32))
                    ok = c64 >= 64
                    return (jnp.where(ok, lo, mid + 1), jnp.where(ok, mid, hi))

                lo, _ = lax.fori_loop(
                    0, 31, wbody,
                    (jnp.zeros((16,), jnp.int32), jnp.zeros((16,), jnp.int32) + r2bits),
                )

                def lchunk(k, acc):
                    for u in range(2):
                        st = pl.multiple_of(k * 32 + u * 16, 16)
                        bits = plsc.bitcast(cd2x[pl.ds(st, 16)], jnp.int32)
                        acc = acc + plsc.all_reduce_population_count(bits < lo)
                    return acc

                cl = lax.fori_loop(0, nch2, lchunk, jnp.zeros((16,), jnp.int32))
                return lo, 64 - cl

            theta, need = lax.cond(
                cnt > 64,
                search,
                lambda: (
                    jnp.zeros((16,), jnp.int32) + (r2bits + 1),
                    jnp.zeros((16,), jnp.int32) + 64,
                ),
            )

            def fchunk(k, carry):
                posv, trankv = carry
                st = pl.multiple_of(k * 16, 16)
                bits = plsc.bitcast(cd2x[pl.ds(st, 16)], jnp.int32)
                less = bits < theta
                tie = bits == theta
                tcs = plsc.cumsum(jnp.where(tie, 1, 0))
                sel = less | (tie & ((trankv + tcs) <= need))
                off = posv + plsc.cumsum(jnp.where(sel, 1, 0)) - 1
                plsc.store_scatter(nbr, [off], cidxx[pl.ds(st, 16)], mask=sel)
                return (
                    posv + plsc.all_reduce_population_count(sel),
                    trankv + plsc.all_reduce_population_count(tie),
                )

            lax.fori_loop(
                0, nch, fchunk,
                (jnp.zeros((16,), jnp.int32), jnp.zeros((16,), jnp.int32)),
            )

            soff = (dg % 8) * (64 * C)
            for j in range(4):
                idxj = nbr[pl.ds(16 * j, 16)]
                rx = plsc.load_gather(pxv, [idxj]) - sxv
                ry = plsc.load_gather(pyv, [idxj]) - syv
                rz = plsc.load_gather(pzv, [idxj]) - szv
                vf = jnp.where((j * 16 + lane) < ncap, 1.0, 0.0)
                rowoff = soff + (j * 16 + lane) * C
                for c in range(F):
                    fvals = plsc.load_gather(ftv, [idxj * F + c])
                    plsc.store_scatter(stg, [rowoff + c], fvals)
                plsc.store_scatter(stg, [rowoff + F], rx)
                plsc.store_scatter(stg, [rowoff + F + 1], ry)
                plsc.store_scatter(stg, [rowoff + F + 2], rz)
                plsc.store_scatter(stg, [rowoff + F + 3], vf)

            @pl.when(dg % 8 == 7)
            def _():
                pltpu.sync_copy(
                    stg, out_h.at[pl.ds((dbase + dg - 7) * 64 * C, 8 * 64 * C)]
                )

        finish(dg0, sxv0, syv0, szv0, cd2, cidx, cntv0)
        finish(dg1, sxv1, syv1, szv1, cd2b, cidxb, cntv1)
        return jnp.int32(0)

    lax.fori_loop(0, dpt // 2, dst_body, jnp.int32(0))


def _sc_neighbors(Pc, S, r, F, C, posx, posy, posz, dstx, dsty, dstz, feat=None):
    total_dst = _B * S
    dpt = total_dst // _NTILES
    r2bits = int(np.float32(r * r).view(np.int32))
    mesh = plsc.VectorSubcoreMesh(core_axis_name="c", subcore_axis_name="s")
    scratch = [
        pltpu.VMEM((Pc,), jnp.float32),
        pltpu.VMEM((Pc,), jnp.float32),
        pltpu.VMEM((Pc,), jnp.float32),
        pltpu.VMEM((dpt,), jnp.float32),
        pltpu.VMEM((dpt,), jnp.float32),
        pltpu.VMEM((dpt,), jnp.float32),
    ]
    if F > 0:
        scratch.append(pltpu.VMEM((Pc * F,), jnp.float32))
    scratch += [
        pltpu.VMEM((Pc + 48,), jnp.float32),
        pltpu.VMEM((Pc + 48,), jnp.int32),
        pltpu.VMEM((Pc + 48,), jnp.float32),
        pltpu.VMEM((Pc + 48,), jnp.int32),
        pltpu.VMEM((80,), jnp.int32),
        pltpu.VMEM((8 * 64 * C,), jnp.float32),
    ]
    fn = pl.kernel(
        partial(_sc_neighbors_body, Pc, S, dpt, r2bits, F, C),
        out_type=jax.ShapeDtypeStruct((total_dst * 64 * C,), jnp.float32),
        mesh=mesh,
        scratch_types=scratch,
        compiler_params=pltpu.CompilerParams(needs_layout_passes=False),
    )
    args = (posx, posy, posz, dstx, dsty, dstz)
    if F > 0:
        args = args + (feat,)
    return fn(*args)


def _edge_mlp_body(F, C, Cout, msg_ref, w0_ref, b0_ref, w1_ref, b1_ref, w2_ref, b2_ref, out_ref):
    x = msg_ref[...]
    v = x[:, F + 3 : F + 4]
    h = jnp.dot(x, w0_ref[...], preferred_element_type=jnp.float32) + b0_ref[...]
    h = jnp.maximum(h, 0.0)
    h = jnp.dot(h, w1_ref[...], preferred_element_type=jnp.float32) + b1_ref[...]
    h = jnp.maximum(h, 0.0)
    h = jnp.dot(h, w2_ref[...], preferred_element_type=jnp.float32) + b2_ref[...]
    h = jnp.where(v > 0.5, h, -jnp.inf)
    te = h.shape[0]
    out_ref[...] = jnp.max(h.reshape(te // 64, 64, Cout), axis=1)


def _edge_mlp(msg, F, C, W0, b0, W1, b1, W2, b2, te):
    E = msg.shape[0]
    C1 = W1.shape[0]
    Cout = W2.shape[1]
    w0p = jnp.zeros((C, C1), jnp.float32).at[: F + 3].set(W0)
    grid = (E // te,)
    return pl.pallas_call(
        partial(_edge_mlp_body, F, C, Cout),
        grid=grid,
        in_specs=[
            pl.BlockSpec((te, C), lambda i: (i, 0)),
            pl.BlockSpec((C, C1), lambda i: (0, 0)),
            pl.BlockSpec((C1,), lambda i: (0,)),
            pl.BlockSpec((C1, C1), lambda i: (0, 0)),
            pl.BlockSpec((C1,), lambda i: (0,)),
            pl.BlockSpec((C1, Cout), lambda i: (0, 0)),
            pl.BlockSpec((Cout,), lambda i: (0,)),
        ],
        out_specs=pl.BlockSpec((te // 64, Cout), lambda i: (i, 0)),
        out_shape=jax.ShapeDtypeStruct((E // 64, Cout), jnp.float32),
    )(msg, w0p, b0, W1, b1, W2, b2)


def _mlp(x, Ws, bs):
    n = len(Ws)
    for j in range(n):
        x = x @ Ws[j] + bs[j]
        if j < n - 1:
            x = jax.nn.relu(x)
    return x


def _fps(pos, S):
    sel0 = jnp.zeros((S,), dtype=jnp.int32)
    d0 = jnp.sum((pos - pos[0]) ** 2, axis=1)

    def body(i, state):
        sel, d = state
        nxt = jnp.argmax(d).astype(jnp.int32)
        sel = sel.at[i].set(nxt)
        d = jnp.minimum(d, jnp.sum((pos - pos[nxt]) ** 2, axis=1))
        return (sel, d)

    sel, _ = jax.lax.fori_loop(1, S, body, (sel0, d0))
    return sel


def _sa_module(x, pos, Pc, S, r, Ws, bs, te):
    pos_c = pos.reshape(_B, Pc, 3)
    pos3 = jnp.transpose(pos_c, (2, 0, 1))
    posd = _fps_pallas(pos3, S)
    pos_dst = jnp.transpose(posd, (1, 2, 0)).reshape(_B * S, 3)
    F = 0 if x is None else x.shape[1]
    C = 8 if x is None else 80
    feat = None if x is None else x.reshape(-1)
    msg = _sc_neighbors(
        Pc, S, r, F, C,
        pos3[0].reshape(-1), pos3[1].reshape(-1), pos3[2].reshape(-1),
        posd[0].reshape(-1), posd[1].reshape(-1), posd[2].reshape(-1),
        feat,
    ).reshape(_B * S * _KNN, C)
    agg = _edge_mlp(msg, F, C, Ws[0], bs[0], Ws[1], bs[1], Ws[2], bs[2], te)
    return agg, pos_dst


def _tail_body(x2_ref, pos2_ref, qp_ref, eps_ref,
               w0a_ref, w0b_ref, b0_ref, w1_ref, b1_ref, w2_ref, b2_ref,
               muw_ref, mub_ref, lvw_ref, lvb_ref,
               d0q_ref, d0s_ref, d0c_ref, d0z_ref,
               d1_ref, d2_ref, d3_ref, d4_ref,
               hq_ref, mu_ref, lv_ref):
    x2 = x2_ref[0]
    pos2 = pos2_ref[0]
    h = (
        jnp.dot(x2, w0a_ref[...], preferred_element_type=jnp.float32)
        + jnp.dot(pos2, w0b_ref[...], preferred_element_type=jnp.float32)
        + b0_ref[...]
    )
    h = jnp.maximum(h, 0.0)
    h = jnp.dot(h, w1_ref[...], preferred_element_type=jnp.float32) + b1_ref[...]
    h = jnp.maximum(h, 0.0)
    h = jnp.dot(h, w2_ref[...], preferred_element_type=jnp.float32) + b2_ref[...]
    g = jnp.max(h, axis=0, keepdims=True)
    mu = jnp.dot(g, muw_ref[...], preferred_element_type=jnp.float32) + mub_ref[...]
    lv = jnp.dot(g, lvw_ref[...], preferred_element_type=jnp.float32) + lvb_ref[...]
    mu_ref[0] = mu
    lv_ref[0] = lv
    z = mu + eps_ref[0] * jnp.exp(0.5 * lv)
    qp = qp_ref[0]
    hz = jnp.dot(z, d0z_ref[...], preferred_element_type=jnp.float32)
    hq = (
        jnp.dot(qp, d0q_ref[...], preferred_element_type=jnp.float32)
        + jnp.dot(jnp.sin(qp), d0s_ref[...], preferred_element_type=jnp.float32)
        + jnp.dot(jnp.cos(qp), d0c_ref[...], preferred_element_type=jnp.float32)
        + hz
    )
    hq = jnp.maximum(hq, 0.0)
    hq = jnp.maximum(jnp.dot(hq, d1_ref[...], preferred_element_type=jnp.float32), 0.0)
    hq = jnp.maximum(jnp.dot(hq, d2_ref[...], preferred_element_type=jnp.float32), 0.0)
    hq = jnp.maximum(jnp.dot(hq, d3_ref[...], preferred_element_type=jnp.float32), 0.0)
    hq_ref[0] = jnp.dot(hq, d4_ref[...], preferred_element_type=jnp.float32)


def _tail_pallas(x2, pos2, qp, eps, sa3_W, sa3_b, mu_W, mu_b, lv_W, lv_b, dec_W):
    S2 = _P // 8
    full = lambda shape: pl.BlockSpec(shape, lambda i: tuple(0 for _ in shape))
    w0a = sa3_W[0][:128]
    w0b = sa3_W[0][128:131]
    d0q = dec_W[0][0:3]
    d0s = dec_W[0][3:6]
    d0c = dec_W[0][6:9]
    d0z = dec_W[0][9:41]
    return pl.pallas_call(
        _tail_body,
        grid=(_B,),
        in_specs=[
            pl.BlockSpec((1, S2, 128), lambda i: (i, 0, 0)),
            pl.BlockSpec((1, S2, 3), lambda i: (i, 0, 0)),
            pl.BlockSpec((1, _Q, 3), lambda i: (i, 0, 0)),
            pl.BlockSpec((1, 1, 32), lambda i: (i, 0, 0)),
            full((128, 128)), full((3, 128)), full((128,)),
            full((128, 256)), full((256,)),
            full((256, 512)), full((512,)),
            full((512, 32)), full((32,)),
            full((512, 32)), full((32,)),
            full((3, 64)), full((3, 64)), full((3, 64)), full((32, 64)),
            full((64, 64)), full((64, 64)), full((64, 64)), full((64, 1)),
        ],
        out_specs=[
            pl.BlockSpec((1, _Q, 1), lambda i: (i, 0, 0)),
            pl.BlockSpec((1, 1, 32), lambda i: (i, 0, 0)),
            pl.BlockSpec((1, 1, 32), lambda i: (i, 0, 0)),
        ],
        out_shape=[
            jax.ShapeDtypeStruct((_B, _Q, 1), jnp.float32),
            jax.ShapeDtypeStruct((_B, 1, 32), jnp.float32),
            jax.ShapeDtypeStruct((_B, 1, 32), jnp.float32),
        ],
    )(
        x2.reshape(_B, S2, 128), pos2.reshape(_B, S2, 3), qp, eps.reshape(_B, 1, 32),
        w0a, w0b, sa3_b[0], sa3_W[1], sa3_b[1], sa3_W[2], sa3_b[2],
        mu_W, mu_b, lv_W, lv_b,
        d0q, d0s, d0c, d0z, dec_W[1], dec_W[2], dec_W[3], dec_W[4],
    )


def kernel(surface_points, batch, query_points, sa1_W0, sa1_b0, sa1_W1, sa1_b1, sa1_W2, sa1_b2, sa2_W0, sa2_b0, sa2_W1, sa2_b1, sa2_W2, sa2_b2, sa3_W0, sa3_b0, sa3_W1, sa3_b1, sa3_W2, sa3_b2, mu_W, mu_b, lv_W, lv_b, dec_W0, dec_W1, dec_W2, dec_W3, dec_W4):
    sp = surface_points
    qp = query_points
    x1, pos1 = _sa_module(None, sp, _P, _P // 2, 0.2, [sa1_W0, sa1_W1, sa1_W2], [sa1_b0, sa1_b1, sa1_b2], te=8192)
    x2, pos2 = _sa_module(x1, pos1, _P // 2, _P // 8, 0.4, [sa2_W0, sa2_W1, sa2_W2], [sa2_b0, sa2_b1, sa2_b2], te=4096)
    eps = jax.random.normal(jax.random.key(1), (_B, 32), dtype=jnp.float32)
    hq, mu, lv = _tail_pallas(
        x2, pos2, qp, eps,
        [sa3_W0, sa3_W1, sa3_W2], [sa3_b0, sa3_b1, sa3_b2],
        mu_W, mu_b, lv_W, lv_b,
        [dec_W0, dec_W1, dec_W2, dec_W3, dec_W4],
    )
    return (hq, mu.reshape(_B, 32), lv.reshape(_B, 32))


# speedup vs baseline: 1.3493x; 1.0951x over previous
import jax
import jax.numpy as jnp
import numpy as np
from jax import lax
from jax.experimental import pallas as pl
from jax.experimental.pallas import tpu as pltpu
from jax.experimental.pallas import tpu_sc as plsc
from functools import partial

_B = 8
_P = 2048
_Q = 1024
_KNN = 64


def _fps_body(S, pos_ref, posd_ref):
    px = pos_ref[0]
    py = pos_ref[1]
    pz = pos_ref[2]
    Pc = px.shape[1]
    sx = px[:, 0:1]
    sy = py[:, 0:1]
    sz = pz[:, 0:1]
    dx = px - sx
    dy = py - sy
    dz = pz - sz
    d = dx * dx + dy * dy + dz * dz
    iota = jax.lax.broadcasted_iota(jnp.int32, (_B, Pc), 1)
    iotaS = jax.lax.broadcasted_iota(jnp.int32, (_B, S), 1)
    big = jnp.int32(Pc)
    col0 = iotaS == 0
    ox = jnp.where(col0, sx, 0.0)
    oy = jnp.where(col0, sy, 0.0)
    oz = jnp.where(col0, sz, 0.0)

    def body(i, carry):
        d, ox, oy, oz = carry
        m = jnp.max(d, axis=1, keepdims=True)
        idxf = jnp.min(jnp.where(d == m, iota, big), axis=1, keepdims=True)
        onehot = (iota == idxf).astype(jnp.float32)
        sx = jnp.sum(px * onehot, axis=1, keepdims=True)
        sy = jnp.sum(py * onehot, axis=1, keepdims=True)
        sz = jnp.sum(pz * onehot, axis=1, keepdims=True)
        coli = iotaS == i
        ox = jnp.where(coli, sx, ox)
        oy = jnp.where(coli, sy, oy)
        oz = jnp.where(coli, sz, oz)
        dx = px - sx
        dy = py - sy
        dz = pz - sz
        d = jnp.minimum(d, dx * dx + dy * dy + dz * dz)
        return (d, ox, oy, oz)

    _, ox, oy, oz = jax.lax.fori_loop(1, S, body, (d, ox, oy, oz))
    posd_ref[0] = ox
    posd_ref[1] = oy
    posd_ref[2] = oz


def _fps_pallas(pos3, S):
    return pl.pallas_call(
        partial(_fps_body, S),
        out_shape=jax.ShapeDtypeStruct((3, _B, S), jnp.float32),
    )(pos3)


_NTILES = 32


def _sc_neighbors_body(Pc, S, dpt, r2bits, F, C, *refs):
    if F > 0:
        (px_h, py_h, pz_h, dx_h, dy_h, dz_h, ft_h, out_h,
         pxv, pyv, pzv, dxv, dyv, dzv, ftv, cd2, cidx, cd2b, cidxb, cd2c, cidxc, cd2d, cidxd, nbr, stg) = refs
    else:
        (px_h, py_h, pz_h, dx_h, dy_h, dz_h, out_h,
         pxv, pyv, pzv, dxv, dyv, dzv, cd2, cidx, cd2b, cidxb, cd2c, cidxc, cd2d, cidxd, nbr, stg) = refs
    wid = lax.axis_index("s") * 2 + lax.axis_index("c")
    dbase = wid * dpt
    b = dbase // S
    pltpu.sync_copy(px_h.at[pl.ds(b * Pc, Pc)], pxv)
    pltpu.sync_copy(py_h.at[pl.ds(b * Pc, Pc)], pyv)
    pltpu.sync_copy(pz_h.at[pl.ds(b * Pc, Pc)], pzv)
    pltpu.sync_copy(dx_h.at[pl.ds(dbase, dpt)], dxv)
    pltpu.sync_copy(dy_h.at[pl.ds(dbase, dpt)], dyv)
    pltpu.sync_copy(dz_h.at[pl.ds(dbase, dpt)], dzv)
    if F > 0:
        pltpu.sync_copy(ft_h.at[pl.ds(b * Pc * F, Pc * F)], ftv)
    z16i = jnp.zeros((16,), jnp.int32)
    z16f = jnp.zeros((16,), jnp.float32)
    for j in range(64 // 16):
        nbr[pl.ds(16 * j, 16)] = z16i

    def zinit(j, _):
        stg[pl.ds(pl.multiple_of(j * 16, 16), 16)] = z16f
        return jnp.int32(0)

    lax.fori_loop(0, 8 * 64 * C // 16, zinit, jnp.int32(0))
    lane = lax.broadcasted_iota(jnp.int32, (16,), 0)
    tru16 = lane < 16
    inf16 = jnp.full((16,), jnp.inf, jnp.float32)
    r2 = jnp.full((16,), float(np.int32(r2bits).view(np.float32)), jnp.float32)

    def dst_body(dl, _):
        dgs = [4 * dl + t for t in range(4)]
        svs = []
        for t in range(4):
            dlv = jnp.zeros((16,), jnp.int32) + dgs[t]
            svs.append((
                plsc.load_gather(dxv, [dlv]),
                plsc.load_gather(dyv, [dlv]),
                plsc.load_gather(dzv, [dlv]),
            ))
        cbufs = [(cd2, cidx), (cd2b, cidxb), (cd2c, cidxc), (cd2d, cidxd)]

        def chunk1(k, carry):
            cnts = list(carry)
            for u in range(2):
                st = pl.multiple_of(k * 32 + u * 16, 16)
                cx = pxv[pl.ds(st, 16)]
                cy = pyv[pl.ds(st, 16)]
                cz = pzv[pl.ds(st, 16)]
                for t in range(4):
                    sxv, syv, szv = svs[t]
                    ax = cx - sxv
                    ay = cy - syv
                    az = cz - szv
                    d2 = ax * ax + ay * ay + az * az
                    m = d2 <= r2
                    off = cnts[t] + plsc.cumsum(jnp.where(m, 1, 0)) - 1
                    plsc.store_scatter(cbufs[t][0], [off], d2, mask=m)
                    plsc.store_scatter(cbufs[t][1], [off], st + lane, mask=m)
                    cnts[t] = cnts[t] + plsc.all_reduce_population_count(m)
            return tuple(cnts)

        cnts = lax.fori_loop(
            0, Pc // 32, chunk1,
            tuple(jnp.zeros((16,), jnp.int32) for _ in range(4)),
        )

        def finish(dg, sxv, syv, szv, cd2x, cidxx, cntv):
            cnt = cntv[0]
            plsc.store_compressed(cd2x.at[pl.ds(cnt, 16)], inf16, mask=tru16)
            plsc.store_compressed(cd2x.at[pl.ds(cnt + 16, 16)], inf16, mask=tru16)
            ncap = jnp.minimum(cnt, 64)
            nch = (cnt + 15) // 16
            nch2 = (cnt + 31) // 32

            def search():
                def wbody(_, c):
                    lo, hi = c
                    mid = (lo + hi) // 2

                    def cchunk(k, acc):
                        for u in range(2):
                            st = pl.multiple_of(k * 32 + u * 16, 16)
                            bits = plsc.bitcast(cd2x[pl.ds(st, 16)], jnp.int32)
                            acc = acc + plsc.all_reduce_population_count(bits <= mid)
                        return acc

                    c64 = lax.fori_loop(0, nch2, cchunk, jnp.zeros((16,), jnp.int32))
                    ok = c64 >= 64
                    return (jnp.where(ok, lo, mid + 1), jnp.where(ok, mid, hi))

                lo, _ = lax.fori_loop(
                    0, 31, wbody,
                    (jnp.zeros((16,), jnp.int32), jnp.zeros((16,), jnp.int32) + r2bits),
                )

                def lchunk(k, acc):
                    for u in range(2):
                        st = pl.multiple_of(k * 32 + u * 16, 16)
                        bits = plsc.bitcast(cd2x[pl.ds(st, 16)], jnp.int32)
                        acc = acc + plsc.all_reduce_population_count(bits < lo)
                    return acc

                cl = lax.fori_loop(0, nch2, lchunk, jnp.zeros((16,), jnp.int32))
                return lo, 64 - cl

            theta, need = lax.cond(
                cnt > 64,
                search,
                lambda: (
                    jnp.zeros((16,), jnp.int32) + (r2bits + 1),
                    jnp.zeros((16,), jnp.int32) + 64,
                ),
            )

            def fchunk(k, carry):
                posv, trankv = carry
                st = pl.multiple_of(k * 16, 16)
                bits = plsc.bitcast(cd2x[pl.ds(st, 16)], jnp.int32)
                less = bits < theta
                tie = bits == theta
                tcs = plsc.cumsum(jnp.where(tie, 1, 0))
                sel = less | (tie & ((trankv + tcs) <= need))
                off = posv + plsc.cumsum(jnp.where(sel, 1, 0)) - 1
                plsc.store_scatter(nbr, [off], cidxx[pl.ds(st, 16)], mask=sel)
                return (
                    posv + plsc.all_reduce_population_count(sel),
                    trankv + plsc.all_reduce_population_count(tie),
                )

            lax.fori_loop(
                0, nch, fchunk,
                (jnp.zeros((16,), jnp.int32), jnp.zeros((16,), jnp.int32)),
            )

            soff = (dg % 8) * (64 * C)
            for j in range(4):
                idxj = nbr[pl.ds(16 * j, 16)]
                rx = plsc.load_gather(pxv, [idxj]) - sxv
                ry = plsc.load_gather(pyv, [idxj]) - syv
                rz = plsc.load_gather(pzv, [idxj]) - szv
                vf = jnp.where((j * 16 + lane) < ncap, 1.0, 0.0)
                rowoff = soff + (j * 16 + lane) * C
                for c in range(F):
                    fvals = plsc.load_gather(ftv, [idxj * F + c])
                    plsc.store_scatter(stg, [rowoff + c], fvals)
                plsc.store_scatter(stg, [rowoff + F], rx)
                plsc.store_scatter(stg, [rowoff + F + 1], ry)
                plsc.store_scatter(stg, [rowoff + F + 2], rz)
                plsc.store_scatter(stg, [rowoff + F + 3], vf)

            @pl.when(dg % 8 == 7)
            def _():
                pltpu.sync_copy(
                    stg, out_h.at[pl.ds((dbase + dg - 7) * 64 * C, 8 * 64 * C)]
                )

        for t in range(4):
            finish(dgs[t], svs[t][0], svs[t][1], svs[t][2],
                   cbufs[t][0], cbufs[t][1], cnts[t])
        return jnp.int32(0)

    lax.fori_loop(0, dpt // 4, dst_body, jnp.int32(0))


def _sc_neighbors(Pc, S, r, F, C, posx, posy, posz, dstx, dsty, dstz, feat=None):
    total_dst = _B * S
    dpt = total_dst // _NTILES
    r2bits = int(np.float32(r * r).view(np.int32))
    mesh = plsc.VectorSubcoreMesh(core_axis_name="c", subcore_axis_name="s")
    scratch = [
        pltpu.VMEM((Pc,), jnp.float32),
        pltpu.VMEM((Pc,), jnp.float32),
        pltpu.VMEM((Pc,), jnp.float32),
        pltpu.VMEM((dpt,), jnp.float32),
        pltpu.VMEM((dpt,), jnp.float32),
        pltpu.VMEM((dpt,), jnp.float32),
    ]
    if F > 0:
        scratch.append(pltpu.VMEM((Pc * F,), jnp.float32))
    scratch += [
        pltpu.VMEM((Pc + 48,), jnp.float32),
        pltpu.VMEM((Pc + 48,), jnp.int32),
        pltpu.VMEM((Pc + 48,), jnp.float32),
        pltpu.VMEM((Pc + 48,), jnp.int32),
        pltpu.VMEM((Pc + 48,), jnp.float32),
        pltpu.VMEM((Pc + 48,), jnp.int32),
        pltpu.VMEM((Pc + 48,), jnp.float32),
        pltpu.VMEM((Pc + 48,), jnp.int32),
        pltpu.VMEM((80,), jnp.int32),
        pltpu.VMEM((8 * 64 * C,), jnp.float32),
    ]
    fn = pl.kernel(
        partial(_sc_neighbors_body, Pc, S, dpt, r2bits, F, C),
        out_type=jax.ShapeDtypeStruct((total_dst * 64 * C,), jnp.float32),
        mesh=mesh,
        scratch_types=scratch,
        compiler_params=pltpu.CompilerParams(needs_layout_passes=False),
    )
    args = (posx, posy, posz, dstx, dsty, dstz)
    if F > 0:
        args = args + (feat,)
    return fn(*args)


def _edge_mlp_body(F, C, Cout, msg_ref, w0_ref, b0_ref, w1_ref, b1_ref, w2_ref, b2_ref, out_ref):
    x = msg_ref[...]
    v = x[:, F + 3 : F + 4]
    h = jnp.dot(x, w0_ref[...], preferred_element_type=jnp.float32) + b0_ref[...]
    h = jnp.maximum(h, 0.0)
    h = jnp.dot(h, w1_ref[...], preferred_element_type=jnp.float32) + b1_ref[...]
    h = jnp.maximum(h, 0.0)
    h = jnp.dot(h, w2_ref[...], preferred_element_type=jnp.float32) + b2_ref[...]
    h = jnp.where(v > 0.5, h, -jnp.inf)
    te = h.shape[0]
    out_ref[...] = jnp.max(h.reshape(te // 64, 64, Cout), axis=1)


def _edge_mlp(msg, F, C, W0, b0, W1, b1, W2, b2, te):
    E = msg.shape[0]
    C1 = W1.shape[0]
    Cout = W2.shape[1]
    w0p = jnp.zeros((C, C1), jnp.float32).at[: F + 3].set(W0)
    grid = (E // te,)
    return pl.pallas_call(
        partial(_edge_mlp_body, F, C, Cout),
        grid=grid,
        in_specs=[
            pl.BlockSpec((te, C), lambda i: (i, 0)),
            pl.BlockSpec((C, C1), lambda i: (0, 0)),
            pl.BlockSpec((C1,), lambda i: (0,)),
            pl.BlockSpec((C1, C1), lambda i: (0, 0)),
            pl.BlockSpec((C1,), lambda i: (0,)),
            pl.BlockSpec((C1, Cout), lambda i: (0, 0)),
            pl.BlockSpec((Cout,), lambda i: (0,)),
        ],
        out_specs=pl.BlockSpec((te // 64, Cout), lambda i: (i, 0)),
        out_shape=jax.ShapeDtypeStruct((E // 64, Cout), jnp.float32),
    )(msg, w0p, b0, W1, b1, W2, b2)


def _mlp(x, Ws, bs):
    n = len(Ws)
    for j in range(n):
        x = x @ Ws[j] + bs[j]
        if j < n - 1:
            x = jax.nn.relu(x)
    return x


def _fps(pos, S):
    sel0 = jnp.zeros((S,), dtype=jnp.int32)
    d0 = jnp.sum((pos - pos[0]) ** 2, axis=1)

    def body(i, state):
        sel, d = state
        nxt = jnp.argmax(d).astype(jnp.int32)
        sel = sel.at[i].set(nxt)
        d = jnp.minimum(d, jnp.sum((pos - pos[nxt]) ** 2, axis=1))
        return (sel, d)

    sel, _ = jax.lax.fori_loop(1, S, body, (sel0, d0))
    return sel


def _sa_module(x, pos, Pc, S, r, Ws, bs, te):
    pos_c = pos.reshape(_B, Pc, 3)
    pos3 = jnp.transpose(pos_c, (2, 0, 1))
    posd = _fps_pallas(pos3, S)
    pos_dst = jnp.transpose(posd, (1, 2, 0)).reshape(_B * S, 3)
    F = 0 if x is None else x.shape[1]
    C = 8 if x is None else 80
    feat = None if x is None else x.reshape(-1)
    msg = _sc_neighbors(
        Pc, S, r, F, C,
        pos3[0].reshape(-1), pos3[1].reshape(-1), pos3[2].reshape(-1),
        posd[0].reshape(-1), posd[1].reshape(-1), posd[2].reshape(-1),
        feat,
    ).reshape(_B * S * _KNN, C)
    agg = _edge_mlp(msg, F, C, Ws[0], bs[0], Ws[1], bs[1], Ws[2], bs[2], te)
    return agg, pos_dst


def _tail_body(x2_ref, pos2_ref, qp_ref, eps_ref,
               w0a_ref, w0b_ref, b0_ref, w1_ref, b1_ref, w2_ref, b2_ref,
               muw_ref, mub_ref, lvw_ref, lvb_ref,
               d0q_ref, d0s_ref, d0c_ref, d0z_ref,
               d1_ref, d2_ref, d3_ref, d4_ref,
               hq_ref, mu_ref, lv_ref):
    x2 = x2_ref[0]
    pos2 = pos2_ref[0]
    h = (
        jnp.dot(x2, w0a_ref[...], preferred_element_type=jnp.float32)
        + jnp.dot(pos2, w0b_ref[...], preferred_element_type=jnp.float32)
        + b0_ref[...]
    )
    h = jnp.maximum(h, 0.0)
    h = jnp.dot(h, w1_ref[...], preferred_element_type=jnp.float32) + b1_ref[...]
    h = jnp.maximum(h, 0.0)
    h = jnp.dot(h, w2_ref[...], preferred_element_type=jnp.float32) + b2_ref[...]
    g = jnp.max(h, axis=0, keepdims=True)
    mu = jnp.dot(g, muw_ref[...], preferred_element_type=jnp.float32) + mub_ref[...]
    lv = jnp.dot(g, lvw_ref[...], preferred_element_type=jnp.float32) + lvb_ref[...]
    mu_ref[0] = mu
    lv_ref[0] = lv
    z = mu + eps_ref[0] * jnp.exp(0.5 * lv)
    qp = qp_ref[0]
    hz = jnp.dot(z, d0z_ref[...], preferred_element_type=jnp.float32)
    hq = (
        jnp.dot(qp, d0q_ref[...], preferred_element_type=jnp.float32)
        + jnp.dot(jnp.sin(qp), d0s_ref[...], preferred_element_type=jnp.float32)
        + jnp.dot(jnp.cos(qp), d0c_ref[...], preferred_element_type=jnp.float32)
        + hz
    )
    hq = jnp.maximum(hq, 0.0)
    hq = jnp.maximum(jnp.dot(hq, d1_ref[...], preferred_element_type=jnp.float32), 0.0)
    hq = jnp.maximum(jnp.dot(hq, d2_ref[...], preferred_element_type=jnp.float32), 0.0)
    hq = jnp.maximum(jnp.dot(hq, d3_ref[...], preferred_element_type=jnp.float32), 0.0)
    hq_ref[0] = jnp.dot(hq, d4_ref[...], preferred_element_type=jnp.float32)


def _tail_pallas(x2, pos2, qp, eps, sa3_W, sa3_b, mu_W, mu_b, lv_W, lv_b, dec_W):
    S2 = _P // 8
    full = lambda shape: pl.BlockSpec(shape, lambda i: tuple(0 for _ in shape))
    w0a = sa3_W[0][:128]
    w0b = sa3_W[0][128:131]
    d0q = dec_W[0][0:3]
    d0s = dec_W[0][3:6]
    d0c = dec_W[0][6:9]
    d0z = dec_W[0][9:41]
    return pl.pallas_call(
        _tail_body,
        grid=(_B,),
        in_specs=[
            pl.BlockSpec((1, S2, 128), lambda i: (i, 0, 0)),
            pl.BlockSpec((1, S2, 3), lambda i: (i, 0, 0)),
            pl.BlockSpec((1, _Q, 3), lambda i: (i, 0, 0)),
            pl.BlockSpec((1, 1, 32), lambda i: (i, 0, 0)),
            full((128, 128)), full((3, 128)), full((128,)),
            full((128, 256)), full((256,)),
            full((256, 512)), full((512,)),
            full((512, 32)), full((32,)),
            full((512, 32)), full((32,)),
            full((3, 64)), full((3, 64)), full((3, 64)), full((32, 64)),
            full((64, 64)), full((64, 64)), full((64, 64)), full((64, 1)),
        ],
        out_specs=[
            pl.BlockSpec((1, _Q, 1), lambda i: (i, 0, 0)),
            pl.BlockSpec((1, 1, 32), lambda i: (i, 0, 0)),
            pl.BlockSpec((1, 1, 32), lambda i: (i, 0, 0)),
        ],
        out_shape=[
            jax.ShapeDtypeStruct((_B, _Q, 1), jnp.float32),
            jax.ShapeDtypeStruct((_B, 1, 32), jnp.float32),
            jax.ShapeDtypeStruct((_B, 1, 32), jnp.float32),
        ],
    )(
        x2.reshape(_B, S2, 128), pos2.reshape(_B, S2, 3), qp, eps.reshape(_B, 1, 32),
        w0a, w0b, sa3_b[0], sa3_W[1], sa3_b[1], sa3_W[2], sa3_b[2],
        mu_W, mu_b, lv_W, lv_b,
        d0q, d0s, d0c, d0z, dec_W[1], dec_W[2], dec_W[3], dec_W[4],
    )


def kernel(surface_points, batch, query_points, sa1_W0, sa1_b0, sa1_W1, sa1_b1, sa1_W2, sa1_b2, sa2_W0, sa2_b0, sa2_W1, sa2_b1, sa2_W2, sa2_b2, sa3_W0, sa3_b0, sa3_W1, sa3_b1, sa3_W2, sa3_b2, mu_W, mu_b, lv_W, lv_b, dec_W0, dec_W1, dec_W2, dec_W3, dec_W4):
    sp = surface_points
    qp = query_points
    x1, pos1 = _sa_module(None, sp, _P, _P // 2, 0.2, [sa1_W0, sa1_W1, sa1_W2], [sa1_b0, sa1_b1, sa1_b2], te=8192)
    x2, pos2 = _sa_module(x1, pos1, _P // 2, _P // 8, 0.4, [sa2_W0, sa2_W1, sa2_W2], [sa2_b0, sa2_b1, sa2_b2], te=4096)
    eps = jax.random.normal(jax.random.key(1), (_B, 32), dtype=jnp.float32)
    hq, mu, lv = _tail_pallas(
        x2, pos2, qp, eps,
        [sa3_W0, sa3_W1, sa3_W2], [sa3_b0, sa3_b1, sa3_b2],
        mu_W, mu_b, lv_W, lv_b,
        [dec_W0, dec_W1, dec_W2, dec_W3, dec_W4],
    )
    return (hq, mu.reshape(_B, 32), lv.reshape(_B, 32))
